# Initial kernel scaffold; baseline (speedup 1.0000x reference)
import jax
import jax.numpy as jnp
from jax.experimental import pallas as pl


def kernel(ent_emb, rel_emb, neigh_w, edge_index, rel_id):
    raise NotImplementedError("write your pallas kernel here")


# speedup vs baseline: 10.8599x; 10.8599x over previous
import functools

import jax
import jax.numpy as jnp
from jax import lax
from jax.experimental import pallas as pl
from jax.experimental.pallas import tpu as pltpu
from jax.experimental.pallas import tpu_sc as plsc

N_NODES = 10000
N_REL = 500
R_PAD = 512
H_DIM = 128
N_EDGES = 320000

N_SC = 2
N_TILES = 16
N_CHUNKS = 4
CHUNK_NODES = N_NODES // N_CHUNKS
CHUNK_WORDS = CHUNK_NODES * R_PAD
TABLE_WORDS = CHUNK_WORDS + 8
EDGES_PER_TILE = N_EDGES // N_TILES
BATCH = 128
N_BATCH = (EDGES_PER_TILE + BATCH - 1) // BATCH
STAGE = N_BATCH * BATCH
TILE_WORDS = CHUNK_WORDS // N_TILES
ZBUF = 8000


def _hist_body(dst_hbm, rel_hbm, out_hbm, dst_v, rel_v, idx_v, ones_v,
               zeros_v, table_sh):
    c = lax.axis_index("c")
    s = lax.axis_index("s")

    base = s * EDGES_PER_TILE
    pltpu.sync_copy(dst_hbm.at[pl.ds(base, EDGES_PER_TILE)],
                    dst_v.at[pl.ds(0, EDGES_PER_TILE)])
    pltpu.sync_copy(rel_hbm.at[pl.ds(base, EDGES_PER_TILE)],
                    rel_v.at[pl.ds(0, EDGES_PER_TILE)])
    neg1 = jnp.full((16,), -1, jnp.int32)
    for k in range(EDGES_PER_TILE, STAGE, 16):
        dst_v[pl.ds(k, 16)] = neg1

    one16 = jnp.ones((16,), jnp.float32)
    for k in range(0, BATCH, 16):
        ones_v[pl.ds(k, 16)] = one16

    zero16 = jnp.zeros((16,), jnp.float32)

    def _zfill(i, carry):
        zeros_v[pl.ds(i * 16, 16)] = zero16
        return carry

    for cc in range(N_CHUNKS // N_SC):
        chunk = c * (N_CHUNKS // N_SC) + cc
        c0 = chunk * CHUNK_NODES

        lax.fori_loop(0, ZBUF // 16, _zfill, 0)
        for z in range(TILE_WORDS // ZBUF):
            pltpu.sync_copy(
                zeros_v, table_sh.at[pl.ds(s * TILE_WORDS + z * ZBUF, ZBUF)])
        plsc.subcore_barrier()

        def _batch(j, carry):
            for k in range(0, BATCH, 16):
                off = j * BATCH + k
                d = dst_v[pl.ds(off, 16)]
                r = rel_v[pl.ds(off, 16)]
                loc = d - c0
                ok = (loc >= 0) & (loc < CHUNK_NODES)
                idx_v[0, pl.ds(k, 16)] = jnp.where(
                    ok, loc * R_PAD + r, CHUNK_WORDS)
            pltpu.sync_copy(ones_v, table_sh.at[idx_v.at[0]], add=True)
            return carry

        lax.fori_loop(0, N_BATCH, _batch, 0)
        plsc.subcore_barrier()

        out_base = chunk * CHUNK_WORDS + s * TILE_WORDS
        for z in range(TILE_WORDS // ZBUF):
            pltpu.sync_copy(
                table_sh.at[pl.ds(s * TILE_WORDS + z * ZBUF, ZBUF)], zeros_v)
            pltpu.sync_copy(
                zeros_v, out_hbm.at[pl.ds(out_base + z * ZBUF, ZBUF)])
        plsc.subcore_barrier()


@functools.cache
def _make_hist():
  return pl.kernel(
    _hist_body,
    out_type=jax.ShapeDtypeStruct((N_NODES * R_PAD,), jnp.float32),
    mesh=plsc.VectorSubcoreMesh(core_axis_name="c", subcore_axis_name="s"),
    scratch_types=[
        pltpu.VMEM((STAGE,), jnp.int32),
        pltpu.VMEM((STAGE,), jnp.int32),
        pltpu.VMEM((1, BATCH), jnp.int32),
        pltpu.VMEM((BATCH,), jnp.float32),
        pltpu.VMEM((ZBUF,), jnp.float32),
        pltpu.VMEM_SHARED((TABLE_WORDS,), jnp.float32),
    ],
  )


def _dense_body(ent_ref, cnt_ref, relp_ref, nw_ref, out_ref):
    ent = ent_ref[...]
    cnt = cnt_ref[...]
    relp = relp_ref[...]
    logits = lax.dot_general(ent, relp, (((1,), (1,)), ((), ())),
                             preferred_element_type=jnp.float32,
                             precision=lax.Precision.HIGHEST)
    mask = cnt > 0.0
    m = jnp.max(jnp.where(mask, logits, -jnp.inf), axis=1, keepdims=True)
    ex = jnp.exp(jnp.where(mask, logits - m, -30.0))
    p = cnt * ex
    ssum = jnp.sum(p, axis=1, keepdims=True)
    w = jnp.where(ssum > 0.0, p / ssum, 0.0)
    neigh = jnp.dot(w, relp, preferred_element_type=jnp.float32,
                    precision=lax.Precision.HIGHEST)
    out_ref[...] = jnp.tanh(jnp.dot(neigh, nw_ref[...],
                                    preferred_element_type=jnp.float32,
                                    precision=lax.Precision.HIGHEST))


BLK = 1000

_dense = pl.pallas_call(
    _dense_body,
    grid=(N_NODES // BLK,),
    in_specs=[
        pl.BlockSpec((BLK, H_DIM), lambda i: (i, 0)),
        pl.BlockSpec((BLK, R_PAD), lambda i: (i, 0)),
        pl.BlockSpec((R_PAD, H_DIM), lambda i: (0, 0)),
        pl.BlockSpec((H_DIM, H_DIM), lambda i: (0, 0)),
    ],
    out_specs=pl.BlockSpec((BLK, H_DIM), lambda i: (i, 0)),
    out_shape=jax.ShapeDtypeStruct((N_NODES, H_DIM), jnp.float32),
)


def kernel(ent_emb, rel_emb, neigh_w, edge_index, rel_id):
    dst = edge_index[1]
    relp = jnp.zeros((R_PAD, H_DIM), jnp.float32).at[:N_REL].set(rel_emb)
    cnt = _make_hist()(dst, rel_id).reshape(N_NODES, R_PAD)
    return _dense(ent_emb, cnt, relp, neigh_w)
